# Initial kernel scaffold; baseline (speedup 1.0000x reference)
import jax
import jax.numpy as jnp
from jax.experimental import pallas as pl


def kernel(target_features_orig, feature_dist_graph, rij_dist_pairs, W_emb, b_emb, W_dist, b_dist, Wh0, Wm0, a0, Wh1, Wm1, a1, Wh2, Wm2, a2, b_scope, start_end_env, l_scope, scope_update, scope_update_lig):
    raise NotImplementedError("write your pallas kernel here")


# speedup vs baseline: 6.1010x; 6.1010x over previous
import functools

import jax
import jax.numpy as jnp
from jax import lax
from jax.experimental import pallas as pl
from jax.experimental.pallas import tpu as pltpu
from jax.experimental.pallas import tpu_sc as plsc

N = 50000
E = 800000
NW = 32
CH = 128
E_PAD = 802816
T_PAD = 51200
PAD_GATHER_ROW = 51000
PAD_SCATTER_ROW = 51100

def _mesh():
    return plsc.VectorSubcoreMesh(core_axis_name="c", subcore_axis_name="s")


def _sc_gather(table, idx, D, B):
    per_w = B // NW
    n_chunks = per_w // CH

    @functools.partial(
        pl.kernel, mesh=_mesh(),
        compiler_params=pltpu.CompilerParams(use_tc_tiling_on_sc=False),
        out_type=jax.ShapeDtypeStruct((B, D), jnp.float32),
        scratch_types=[
            pltpu.VMEM((CH,), jnp.int32),
            pltpu.VMEM((CH, D), jnp.float32),
            pltpu.SemaphoreType.DMA,
        ],
    )
    def k(table_hbm, idx_hbm, out_hbm, idx_v, rows_v, sem):
        wid = lax.axis_index("s") * 2 + lax.axis_index("c")
        base = wid * per_w

        def body(j, carry):
            off = base + j * CH
            pltpu.sync_copy(idx_hbm.at[pl.ds(off, CH)], idx_v)
            pltpu.async_copy(table_hbm.at[idx_v], rows_v, sem).wait()
            pltpu.sync_copy(rows_v, out_hbm.at[pl.ds(off, CH)])
            return carry

        lax.fori_loop(0, n_chunks, body, 0)

    return k(table, idx)


def _sc_scatter_add(vals, idx, zeros_slab):
    per_w = E_PAD // NW
    n_chunks = per_w // CH
    rows_per_sub = T_PAD // 16

    @functools.partial(
        pl.kernel, mesh=_mesh(),
        compiler_params=pltpu.CompilerParams(use_tc_tiling_on_sc=False),
        out_type=jax.ShapeDtypeStruct((2, T_PAD, 32), jnp.float32),
        scratch_types=[
            pltpu.VMEM((CH,), jnp.int32),
            pltpu.VMEM((CH, 32), jnp.float32),
            pltpu.VMEM_SHARED((T_PAD, 32), jnp.float32),
        ],
    )
    def k(vals_hbm, idx_hbm, zeros_hbm, out_hbm, idx_v, v_v, acc_sh):
        cid = lax.axis_index("c")
        sid = lax.axis_index("s")
        wid = sid * 2 + cid
        base = wid * per_w
        my_rows = sid * rows_per_sub

        pltpu.sync_copy(zeros_hbm, acc_sh.at[pl.ds(my_rows, rows_per_sub)])
        plsc.subcore_barrier()

        def body(j, carry):
            off = base + j * CH
            pltpu.sync_copy(idx_hbm.at[pl.ds(off, CH)], idx_v)
            pltpu.sync_copy(vals_hbm.at[pl.ds(off, CH)], v_v)
            pltpu.sync_copy(v_v, acc_sh.at[idx_v], add=True)
            return carry

        lax.fori_loop(0, n_chunks, body, 0)
        plsc.subcore_barrier()
        pltpu.sync_copy(acc_sh.at[pl.ds(my_rows, rows_per_sub)],
                        out_hbm.at[cid, pl.ds(my_rows, rows_per_sub)])

    return k(vals, idx, zeros_slab)


def _pad_table(tbl):
    D = tbl.shape[1]
    return jnp.concatenate(
        [tbl, jnp.zeros((T_PAD - tbl.shape[0], D), tbl.dtype)], axis=0)


def _tc_matmul_bias(x, w, b, R):
    M, K = x.shape
    F = w.shape[1]

    def body(x_ref, w_ref, b_ref, o_ref):
        o_ref[...] = x_ref[...] @ w_ref[...] + b_ref[...]

    return pl.pallas_call(
        body,
        grid=(M // R,),
        in_specs=[
            pl.BlockSpec((R, K), lambda i: (i, 0)),
            pl.BlockSpec((K, F), lambda i: (0, 0)),
            pl.BlockSpec((1, F), lambda i: (0, 0)),
        ],
        out_specs=pl.BlockSpec((R, F), lambda i: (i, 0)),
        out_shape=jax.ShapeDtypeStruct((M, F), jnp.float32),
    )(x, w, b.reshape(1, F))


def _tc_head(src_f, msg, Wh, Wm, a):
    M, d_in = src_f.shape
    R = 2048

    def body(s_ref, m_ref, wh_ref, wm_ref, a_ref, o_ref):
        hs = s_ref[...] @ wh_ref[...]
        hm = m_ref[...] @ wm_ref[...]
        e = hs @ a_ref[0:16, :] + hm @ a_ref[16:32, :]
        e = jnp.where(e >= 0, e, 0.2 * e)
        ex = jnp.exp(e)
        o_ref[...] = jnp.concatenate(
            [ex * hm, jnp.broadcast_to(ex, (R, 16))], axis=1)

    return pl.pallas_call(
        body,
        grid=(M // R,),
        in_specs=[
            pl.BlockSpec((R, d_in), lambda i: (i, 0)),
            pl.BlockSpec((R, d_in), lambda i: (i, 0)),
            pl.BlockSpec((d_in, 16), lambda i: (0, 0)),
            pl.BlockSpec((d_in, 16), lambda i: (0, 0)),
            pl.BlockSpec((32, 1), lambda i: (0, 0)),
        ],
        out_specs=pl.BlockSpec((R, 32), lambda i: (i, 0)),
        out_shape=jax.ShapeDtypeStruct((M, 32), jnp.float32),
    )(src_f, msg, Wh, Wm, a.reshape(32, 1))


def _tc_headout(sums):
    R = 1024

    def body(s_ref, o_ref):
        s = s_ref[0] + s_ref[1]
        num = s[:, 0:16]
        den = s[:, 16:17]
        h = num / (den + 1e-9)
        o_ref[...] = jnp.where(h > 0, h, jnp.exp(h) - 1.0)

    return pl.pallas_call(
        body,
        grid=(T_PAD // R,),
        in_specs=[pl.BlockSpec((2, R, 32), lambda i: (0, i, 0))],
        out_specs=pl.BlockSpec((R, 16), lambda i: (i, 0)),
        out_shape=jax.ShapeDtypeStruct((T_PAD, 16), jnp.float32),
    )(sums)


def _tc_avg(x, y):
    M, D = x.shape
    R = 2048

    def body(x_ref, y_ref, o_ref):
        o_ref[...] = 0.5 * (x_ref[...] + y_ref[...])

    return pl.pallas_call(
        body,
        grid=(M // R,),
        in_specs=[pl.BlockSpec((R, D), lambda i: (i, 0)),
                  pl.BlockSpec((R, D), lambda i: (i, 0))],
        out_specs=pl.BlockSpec((R, D), lambda i: (i, 0)),
        out_shape=jax.ShapeDtypeStruct((M, D), jnp.float32),
    )(x, y)


def _tc_readout(cmp_enc):
    def body(c_ref, o_ref):
        o_ref[...] = jnp.sum(c_ref[...], axis=1)

    return pl.pallas_call(
        body,
        in_specs=[pl.BlockSpec((256, 32, 64), lambda: (0, 0, 0))],
        out_specs=pl.BlockSpec((256, 64), lambda: (0, 0)),
        out_shape=jax.ShapeDtypeStruct((256, 64), jnp.float32),
    )(cmp_enc)


def _pad_idx(idx):
    return jnp.concatenate(
        [idx.astype(jnp.int32),
         jnp.full((E_PAD - E,), PAD_GATHER_ROW, jnp.int32)])


def kernel(target_features_orig, feature_dist_graph, rij_dist_pairs,
           W_emb, b_emb, W_dist, b_dist,
           Wh0, Wm0, a0, Wh1, Wm1, a1, Wh2, Wm2, a2,
           b_scope, start_end_env, l_scope, scope_update, scope_update_lig):
    se = _pad_idx(start_end_env)
    su = _pad_idx(scope_update)
    sul = _pad_idx(scope_update_lig)
    seg = jnp.concatenate(
        [b_scope.astype(jnp.int32),
         jnp.full((E_PAD - E,), PAD_SCATTER_ROW, jnp.int32)])
    zeros_slab = jnp.zeros((T_PAD // 16, 32), jnp.float32)

    tf0 = _tc_matmul_bias(target_features_orig, W_emb, b_emb, R=2000)
    feats = jnp.concatenate([jnp.zeros((1, 16), jnp.float32), tf0], axis=0)

    x_dist = jnp.concatenate(
        [feature_dist_graph, rij_dist_pairs[:, None]], axis=1)
    x_dist = jnp.concatenate(
        [x_dist, jnp.zeros((E_PAD - E, 16), jnp.float32)], axis=0)
    msg = _tc_matmul_bias(x_dist, W_dist, b_dist, R=2048)

    heads = [(Wh0, Wm0, a0), (Wh1, Wm1, a1), (Wh2, Wm2, a2)]
    for i, (Wh, Wm, a) in enumerate(heads):
        D = 16 * (i + 1)
        tbl = _pad_table(feats)
        src_f = _sc_gather(tbl, se, D, E_PAD)
        V = _tc_head(src_f, msg, Wh, Wm, a)
        sums = _sc_scatter_add(V, seg, zeros_slab)
        new = _tc_headout(sums)[:N + 1]
        feats = jnp.concatenate([new, feats], axis=1)
        if i < 2:
            tbl2 = _pad_table(feats)
            g1 = _sc_gather(tbl2, su, D + 16, E_PAD)
            g2 = _sc_gather(tbl2, sul, D + 16, E_PAD)
            msg = _tc_avg(g1, g2)

    cmp_rows = _sc_gather(_pad_table(feats),
                          l_scope.astype(jnp.int32).reshape(-1), 64, 256 * 32)
    return _tc_readout(cmp_rows.reshape(256, 32, 64))
